# baseline (speedup 1.0000x reference)
import functools

import jax
import jax.numpy as jnp
from jax import lax
from jax.experimental import pallas as pl
from jax.experimental.pallas import tpu as pltpu
from jax.experimental.pallas import tpu_sc as plsc

_NUM_CORES = 2
_NUM_SUBCORES = 16
_NUM_WORKERS = _NUM_CORES * _NUM_SUBCORES
_LANES = 16

_GATHER_CHUNK = 128


def _make_sc_lookup(B, L, V, D):
  mesh = plsc.VectorSubcoreMesh(core_axis_name="c", subcore_axis_name="s")
  b_per_w = B // _NUM_WORKERS

  @functools.partial(
      pl.kernel,
      out_type=jax.ShapeDtypeStruct((B, L, D), jnp.float32),
      mesh=mesh,
      scratch_types=[
          pltpu.VMEM((L, D), jnp.float32),
          pltpu.VMEM((L,), jnp.int32),
          pltpu.VMEM((L, D), jnp.float32),
          pltpu.SemaphoreType.DMA,
      ],
  )
  def lookup(x_hbm, pe_hbm, emb_hbm, out_hbm, pe_v, idx_v, rows_v, sem):
    wid = lax.axis_index("s") * _NUM_CORES + lax.axis_index("c")
    pltpu.sync_copy(pe_hbm, pe_v)

    def per_batch(nb, _):
      b = wid * b_per_w + nb
      pltpu.sync_copy(x_hbm.at[b], idx_v)
      n_full = L // _GATHER_CHUNK
      copies = []
      for g in range(n_full):
        lo = g * _GATHER_CHUNK
        copies.append(pltpu.async_copy(
            emb_hbm.at[idx_v.at[pl.ds(lo, _GATHER_CHUNK)]],
            rows_v.at[pl.ds(lo, _GATHER_CHUNK)], sem))
      rem = L - n_full * _GATHER_CHUNK
      if rem:
        lo = n_full * _GATHER_CHUNK
        copies.append(pltpu.async_copy(
            emb_hbm.at[idx_v.at[pl.ds(lo, rem)]],
            rows_v.at[pl.ds(lo, rem)], sem))
      for cp in copies:
        cp.wait()

      def add_row(r, _):
        def add_chunk(c, _):
          sl = pl.ds(c * _LANES, _LANES)
          rows_v[r, sl] = rows_v[r, sl] + pe_v[r, sl]
          return 0
        return lax.fori_loop(0, D // _LANES, add_chunk, 0)
      lax.fori_loop(0, L, add_row, 0)

      pltpu.sync_copy(rows_v, out_hbm.at[b])
      return 0

    lax.fori_loop(0, b_per_w, per_batch, 0)

  return lookup


def kernel(x, offset, emb, pe):
  B, L = x.shape
  V, D = emb.shape
  pe_s = lax.dynamic_slice_in_dim(pe, offset, L, axis=0)
  return _make_sc_lookup(B, L, V, D)(x, pe_s, emb)


# speedup vs baseline: 1.7019x; 1.7019x over previous
import functools

import jax
import jax.numpy as jnp
from jax import lax
from jax.experimental import pallas as pl
from jax.experimental.pallas import tpu as pltpu
from jax.experimental.pallas import tpu_sc as plsc

_NUM_CORES = 2
_NUM_SUBCORES = 16
_NUM_WORKERS = _NUM_CORES * _NUM_SUBCORES
_LANES = 16
_NBUF = 4


def _make_sc_lookup(B, L, V, D):
  mesh = plsc.VectorSubcoreMesh(core_axis_name="c", subcore_axis_name="s")
  b_per_w = B // _NUM_WORKERS
  ch0 = ((L // 2) + 7) // 8 * 8
  ch1 = L - ch0
  assert 0 < ch1 <= 128 and ch0 <= 128 and ch0 % 8 == 0
  n_chunks = 2 * b_per_w
  chunk_sz = (ch0, ch1)
  chunk_off = (0, ch0)

  @functools.partial(
      pl.kernel,
      out_type=jax.ShapeDtypeStruct((B, L, D), jnp.float32),
      mesh=mesh,
      scratch_types=[
          pltpu.VMEM((L, D), jnp.float32),
          pltpu.VMEM((b_per_w * L,), jnp.int32),
      ] + [pltpu.VMEM((ch0, D), jnp.float32) for _ in range(_NBUF)]
        + [pltpu.SemaphoreType.DMA for _ in range(2 * _NBUF)],
  )
  def lookup(x_hbm, pe_hbm, emb_hbm, out_hbm, pe_v, idx_v, *bufs_sems):
    rows = bufs_sems[:_NBUF]
    gsem = bufs_sems[_NBUF:2 * _NBUF]
    ssem = bufs_sems[2 * _NBUF:]
    wid = lax.axis_index("s") * _NUM_CORES + lax.axis_index("c")
    base = wid * b_per_w
    pltpu.sync_copy(x_hbm.at[pl.ds(base * L, b_per_w * L)], idx_v)
    pltpu.sync_copy(pe_hbm, pe_v)

    def gather_start(nb, p):
      sz = chunk_sz[p & 1]
      pltpu.async_copy(
          emb_hbm.at[idx_v.at[pl.ds(nb * L + chunk_off[p & 1], sz)]],
          rows[p].at[pl.ds(0, sz)], gsem[p])

    def gather_drain(p):
      sz = chunk_sz[p & 1]
      pltpu.make_async_copy(
          emb_hbm.at[pl.ds(0, sz)], rows[p].at[pl.ds(0, sz)], gsem[p]).wait()

    def add_pe(p):
      sz = chunk_sz[p & 1]
      off = chunk_off[p & 1]
      buf = rows[p]

      def add_row(r, _):
        for d in range(D // _LANES):
          sl = pl.ds(d * _LANES, _LANES)
          buf[r, sl] = buf[r, sl] + pe_v[off + r, sl]
        return 0
      lax.fori_loop(0, sz, add_row, 0)

    def store_start(nb, p):
      sz = chunk_sz[p & 1]
      pltpu.async_copy(
          rows[p].at[pl.ds(0, sz)],
          out_hbm.at[base + nb, pl.ds(chunk_off[p & 1], sz)], ssem[p])

    def store_drain(p):
      sz = chunk_sz[p & 1]
      pltpu.make_async_copy(
          emb_hbm.at[pl.ds(0, sz)], rows[p].at[pl.ds(0, sz)], ssem[p]).wait()

    gather_start(0, 0)
    gather_start(0, 1)

    @pl.loop(0, n_chunks, step=_NBUF)
    def _(c0):
      for j in range(_NBUF):
        c = c0 + j
        nb = c0 // 2 + (j // 2)
        gather_drain(j)
        add_pe(j)
        store_start(nb, j)
        q = (j + 2) % _NBUF
        nb_pre = c0 // 2 + (j + 2) // 2

        @pl.when(c >= 2)
        def _():
          store_drain(q)

        @pl.when(c + 2 < n_chunks)
        def _():
          gather_start(nb_pre, q)

    store_drain(_NBUF - 2)
    store_drain(_NBUF - 1)

  return lookup


def kernel(x, offset, emb, pe):
  B, L = x.shape
  V, D = emb.shape
  pe_s = lax.dynamic_slice_in_dim(pe, offset, L, axis=0)
  return _make_sc_lookup(B, L, V, D)(x.reshape(-1), pe_s, emb)


# speedup vs baseline: 1.8546x; 1.0897x over previous
import functools

import jax
import jax.numpy as jnp
from jax import lax
from jax.experimental import pallas as pl
from jax.experimental.pallas import tpu as pltpu
from jax.experimental.pallas import tpu_sc as plsc

_NUM_CORES = 2
_NUM_SUBCORES = 16
_NUM_WORKERS = _NUM_CORES * _NUM_SUBCORES
_LANES = 16
_NBUF = 4


def _make_sc_lookup(B, L, V, D):
  mesh = plsc.VectorSubcoreMesh(core_axis_name="c", subcore_axis_name="s")
  b_per_w = B // _NUM_WORKERS
  ch0 = ((L // 2) + 7) // 8 * 8
  ch1 = L - ch0
  assert 0 < ch1 <= 128 and ch0 <= 128 and ch0 % 8 == 0
  n_chunks = 2 * b_per_w
  chunk_sz = (ch0, ch1)
  chunk_off = (0, ch0)

  @functools.partial(
      pl.kernel,
      out_type=jax.ShapeDtypeStruct((B, L, D), jnp.float32),
      mesh=mesh,
      scratch_types=[
          pltpu.VMEM_SHARED((L, D), jnp.float32),
          pltpu.VMEM((b_per_w * L,), jnp.int32),
      ] + [pltpu.VMEM((ch0, D), jnp.float32) for _ in range(_NBUF)]
        + [pltpu.SemaphoreType.DMA for _ in range(2 * _NBUF)],
  )
  def lookup(x_hbm, pe_hbm, emb_hbm, out_hbm, pe_v, idx_v, *bufs_sems):
    rows = bufs_sems[:_NBUF]
    gsem = bufs_sems[_NBUF:2 * _NBUF]
    ssem = bufs_sems[2 * _NBUF:]
    wid = lax.axis_index("s") * _NUM_CORES + lax.axis_index("c")
    base = wid * b_per_w
    pltpu.sync_copy(x_hbm.at[pl.ds(base * L, b_per_w * L)], idx_v)

    @pl.when(lax.axis_index("s") == 0)
    def _():
      pltpu.sync_copy(pe_hbm, pe_v)
    plsc.subcore_barrier()

    def gather_start(nb, p):
      sz = chunk_sz[p & 1]
      pltpu.sync_copy(pe_v.at[pl.ds(chunk_off[p & 1], sz)],
                      rows[p].at[pl.ds(0, sz)])
      pltpu.async_copy(
          emb_hbm.at[idx_v.at[pl.ds(nb * L + chunk_off[p & 1], sz)]],
          rows[p].at[pl.ds(0, sz)], gsem[p], add=True)

    def gather_drain(p):
      sz = chunk_sz[p & 1]
      pltpu.make_async_copy(
          emb_hbm.at[pl.ds(0, sz)], rows[p].at[pl.ds(0, sz)], gsem[p]).wait()

    def add_pe(p):
      sz = chunk_sz[p & 1]
      off = chunk_off[p & 1]
      buf = rows[p]

      def add_row(r, _):
        for d in range(D // _LANES):
          sl = pl.ds(d * _LANES, _LANES)
          buf[r, sl] = buf[r, sl] + pe_v[off + r, sl]
        return 0
      lax.fori_loop(0, sz, add_row, 0)

    def store_start(nb, p):
      sz = chunk_sz[p & 1]
      pltpu.async_copy(
          rows[p].at[pl.ds(0, sz)],
          out_hbm.at[base + nb, pl.ds(chunk_off[p & 1], sz)], ssem[p])

    def store_drain(p):
      sz = chunk_sz[p & 1]
      pltpu.make_async_copy(
          emb_hbm.at[pl.ds(0, sz)], rows[p].at[pl.ds(0, sz)], ssem[p]).wait()

    gather_start(0, 0)
    gather_start(0, 1)

    @pl.loop(0, n_chunks, step=_NBUF)
    def _(c0):
      for j in range(_NBUF):
        c = c0 + j
        nb = c0 // 2 + (j // 2)
        gather_drain(j)
        store_start(nb, j)
        q = (j + 2) % _NBUF
        nb_pre = c0 // 2 + (j + 2) // 2

        @pl.when(c >= 2)
        def _():
          store_drain(q)

        @pl.when(c + 2 < n_chunks)
        def _():
          gather_start(nb_pre, q)

    store_drain(_NBUF - 2)
    store_drain(_NBUF - 1)

  return lookup


def kernel(x, offset, emb, pe):
  B, L = x.shape
  V, D = emb.shape
  pe_s = lax.dynamic_slice_in_dim(pe, offset, L, axis=0)
  return _make_sc_lookup(B, L, V, D)(x.reshape(-1), pe_s, emb)


# speedup vs baseline: 1.9917x; 1.0739x over previous
import functools

import jax
import jax.numpy as jnp
from jax import lax
from jax.experimental import pallas as pl
from jax.experimental.pallas import tpu as pltpu
from jax.experimental.pallas import tpu_sc as plsc

_NUM_CORES = 2
_NUM_SUBCORES = 16
_NUM_WORKERS = _NUM_CORES * _NUM_SUBCORES
_NBUF = 8
_PD = 4


def _make_sc_lookup(B, L, V, D):
  mesh = plsc.VectorSubcoreMesh(core_axis_name="c", subcore_axis_name="s")
  b_per_w = B // _NUM_WORKERS
  ch0 = ((L // 2) + 7) // 8 * 8
  ch1 = L - ch0
  assert 0 < ch1 <= 128 and ch0 <= 128 and ch0 % 8 == 0
  n_chunks = 2 * b_per_w
  assert n_chunks % _NBUF == 0 and _PD % 2 == 0 and _PD < _NBUF
  chunk_sz = (ch0, ch1)
  chunk_off = (0, ch0)

  @functools.partial(
      pl.kernel,
      out_type=jax.ShapeDtypeStruct((B, L, D), jnp.float32),
      mesh=mesh,
      scratch_types=[
          pltpu.VMEM_SHARED((L, D), jnp.float32),
          pltpu.VMEM((b_per_w * L,), jnp.int32),
      ] + [pltpu.VMEM((ch0, D), jnp.float32) for _ in range(_NBUF)]
        + [pltpu.SemaphoreType.DMA for _ in range(2 * _NBUF)],
  )
  def lookup(x_hbm, pe_hbm, emb_hbm, out_hbm, pe_sh, idx_v, *bufs_sems):
    rows = bufs_sems[:_NBUF]
    gsem = bufs_sems[_NBUF:2 * _NBUF]
    ssem = bufs_sems[2 * _NBUF:]
    wid = lax.axis_index("s") * _NUM_CORES + lax.axis_index("c")
    base = wid * b_per_w
    pltpu.sync_copy(x_hbm.at[pl.ds(base * L, b_per_w * L)], idx_v)

    @pl.when(lax.axis_index("s") == 0)
    def _():
      pltpu.sync_copy(pe_hbm, pe_sh)
    plsc.subcore_barrier()

    def gather_start(nb, p):
      sz = chunk_sz[p & 1]
      pltpu.sync_copy(pe_sh.at[pl.ds(chunk_off[p & 1], sz)],
                      rows[p].at[pl.ds(0, sz)])
      pltpu.async_copy(
          emb_hbm.at[idx_v.at[pl.ds(nb * L + chunk_off[p & 1], sz)]],
          rows[p].at[pl.ds(0, sz)], gsem[p], add=True)

    def gather_drain(p):
      sz = chunk_sz[p & 1]
      pltpu.make_async_copy(
          emb_hbm.at[pl.ds(0, sz)], rows[p].at[pl.ds(0, sz)], gsem[p]).wait()

    def store_start(nb, p):
      sz = chunk_sz[p & 1]
      pltpu.async_copy(
          rows[p].at[pl.ds(0, sz)],
          out_hbm.at[base + nb, pl.ds(chunk_off[p & 1], sz)], ssem[p])

    def store_drain(p):
      sz = chunk_sz[p & 1]
      pltpu.make_async_copy(
          emb_hbm.at[pl.ds(0, sz)], rows[p].at[pl.ds(0, sz)], ssem[p]).wait()

    for c in range(_PD):
      gather_start(c // 2, c)

    @pl.loop(0, n_chunks, step=_NBUF)
    def _(c0):
      for j in range(_NBUF):
        c = c0 + j
        nb = c0 // 2 + (j // 2)
        gather_drain(j)
        store_start(nb, j)
        q = (j + _PD) % _NBUF
        nb_pre = c0 // 2 + (j + _PD) // 2

        @pl.when(c >= _NBUF - _PD)
        def _():
          store_drain(q)

        @pl.when(c + _PD < n_chunks)
        def _():
          gather_start(nb_pre, q)

    for p in range(_NBUF - _PD, _NBUF):
      store_drain(p)

  return lookup


def kernel(x, offset, emb, pe):
  B, L = x.shape
  V, D = emb.shape
  pe_s = lax.dynamic_slice_in_dim(pe, offset, L, axis=0)
  return _make_sc_lookup(B, L, V, D)(x.reshape(-1), pe_s, emb)
